# Initial kernel scaffold; baseline (speedup 1.0000x reference)
import jax
import jax.numpy as jnp
from jax.experimental import pallas as pl


def kernel(x, edge_index, edge_weight, W_in, b_in, W_out, b_out):
    raise NotImplementedError("write your pallas kernel here")


# speedup vs baseline: 3.6574x; 3.6574x over previous
import functools

import jax
import jax.numpy as jnp
from jax import lax
from jax.experimental import pallas as pl
from jax.experimental.pallas import tpu as pltpu
from jax.experimental.pallas import tpu_sc as plsc

N = 10000
E = 320000
IN_CH = 128
D = 64
K = 3
ALPHA = 0.1

NC = 2
NS = 16
NW = NC * NS
L = 16

N_PAD = 10240
RPT = N_PAD // NS
B = 128
CH = 79
E_TILE = B * CH
E_PAD = E_TILE * NW

ROWBLK = 1024


def _mesh():
    return plsc.VectorSubcoreMesh(
        core_axis_name="c", subcore_axis_name="s",
        num_cores=NC, num_subcores=NS)


@functools.partial(
    pl.kernel,
    out_type=jax.ShapeDtypeStruct((NC, 2, N_PAD), jnp.float32),
    mesh=_mesh(),
    compiler_params=pltpu.CompilerParams(use_tc_tiling_on_sc=False),
    scratch_types=[
        pltpu.VMEM((CH, B), jnp.int32),
        pltpu.VMEM((CH, B), jnp.int32),
        pltpu.VMEM((B,), jnp.float32),
        pltpu.VMEM((RPT,), jnp.float32),
        pltpu.VMEM_SHARED((N_PAD,), jnp.float32),
        pltpu.VMEM_SHARED((N_PAD,), jnp.float32),
    ],
)
def _deg_kernel(src_h, dst_h, out_h, src_v, dst_v, ones_v, zero_v,
                dego_sh, degi_sh):
    cid = lax.axis_index("c")
    sid = lax.axis_index("s")
    wid = cid * NS + sid

    def zi(i, c):
        zero_v[pl.ds(i * L, L)] = jnp.zeros((L,), jnp.float32)
        return c
    lax.fori_loop(0, RPT // L, zi, 0)

    def oi(i, c):
        ones_v[pl.ds(i * L, L)] = jnp.ones((L,), jnp.float32)
        return c
    lax.fori_loop(0, B // L, oi, 0)

    pltpu.sync_copy(zero_v, dego_sh.at[pl.ds(sid * RPT, RPT)])
    pltpu.sync_copy(zero_v, degi_sh.at[pl.ds(sid * RPT, RPT)])
    pltpu.sync_copy(src_h.at[wid], src_v)
    pltpu.sync_copy(dst_h.at[wid], dst_v)
    plsc.subcore_barrier()

    def body(j, c):
        pltpu.sync_copy(ones_v, dego_sh.at[src_v.at[j]], add=True)
        pltpu.sync_copy(ones_v, degi_sh.at[dst_v.at[j]], add=True)
        return c
    lax.fori_loop(0, CH, body, 0)

    plsc.subcore_barrier()
    sl = pl.ds(sid * RPT, RPT)
    pltpu.sync_copy(dego_sh.at[sl], out_h.at[cid, 0, sl])
    pltpu.sync_copy(degi_sh.at[sl], out_h.at[cid, 1, sl])


@functools.partial(
    pl.kernel,
    out_type=jax.ShapeDtypeStruct((NW, CH, B), jnp.float32),
    mesh=_mesh(),
    compiler_params=pltpu.CompilerParams(use_tc_tiling_on_sc=False),
    scratch_types=[
        pltpu.VMEM((CH, B), jnp.int32),
        pltpu.VMEM((CH, B), jnp.int32),
        pltpu.VMEM((CH, B), jnp.float32),
        pltpu.VMEM((CH, B), jnp.float32),
        pltpu.VMEM((B,), jnp.float32),
        pltpu.VMEM((B,), jnp.float32),
        pltpu.SemaphoreType.DMA,
    ],
)
def _coef_kernel(src_h, dst_h, ew_h, no_h, ni_h, out_h,
                 src_v, dst_v, ew_v, coef_v, nog_v, nig_v, sem):
    cid = lax.axis_index("c")
    sid = lax.axis_index("s")
    wid = cid * NS + sid
    pltpu.sync_copy(src_h.at[wid], src_v)
    pltpu.sync_copy(dst_h.at[wid], dst_v)
    pltpu.sync_copy(ew_h.at[wid], ew_v)

    def body(j, c):
        pltpu.async_copy(no_h.at[src_v.at[j]], nog_v, sem).wait()
        pltpu.async_copy(ni_h.at[dst_v.at[j]], nig_v, sem).wait()
        for g in range(B // L):
            sl = pl.ds(g * L, L)
            coef_v[j, sl] = nog_v[sl] * ew_v[j, sl] * nig_v[sl]
        return c
    lax.fori_loop(0, CH, body, 0)
    pltpu.sync_copy(coef_v, out_h.at[wid])


@functools.partial(
    pl.kernel,
    out_type=jax.ShapeDtypeStruct((NC, N_PAD, D), jnp.float32),
    mesh=_mesh(),
    compiler_params=pltpu.CompilerParams(use_tc_tiling_on_sc=False),
    scratch_types=[
        pltpu.VMEM((CH, B), jnp.int32),
        pltpu.VMEM((CH, B), jnp.int32),
        pltpu.VMEM((CH, B), jnp.float32),
        pltpu.VMEM((B, D), jnp.float32),
        pltpu.VMEM((B, D), jnp.float32),
        pltpu.VMEM_SHARED((N_PAD, D), jnp.float32),
        pltpu.SemaphoreType.DMA,
    ],
)
def _prop_kernel(h_h, src_h, dst_h, coef_h, out_h,
                 src_v, dst_v, coef_v, rows_v, zero_v, agg_sh, sem):
    cid = lax.axis_index("c")
    sid = lax.axis_index("s")
    wid = cid * NS + sid

    def zi(i, c):
        zero_v[i // (D // L), pl.ds((i % (D // L)) * L, L)] = (
            jnp.zeros((L,), jnp.float32))
        return c
    lax.fori_loop(0, B * D // L, zi, 0)

    def zc(i, c):
        pltpu.sync_copy(zero_v, agg_sh.at[pl.ds(sid * RPT + i * B, B)])
        return c
    lax.fori_loop(0, RPT // B, zc, 0)

    pltpu.sync_copy(src_h.at[wid], src_v)
    pltpu.sync_copy(dst_h.at[wid], dst_v)
    pltpu.sync_copy(coef_h.at[wid], coef_v)
    plsc.subcore_barrier()

    def body(j, c):
        pltpu.async_copy(h_h.at[src_v.at[j]], rows_v, sem).wait()

        def scale(g, cc):
            cvec = coef_v[j, pl.ds(g * L, L)]
            for m in range(L):
                e = g * L + m
                s = cvec[m]
                for k in range(D // L):
                    rows_v[e, pl.ds(k * L, L)] = (
                        rows_v[e, pl.ds(k * L, L)] * s)
            return cc
        lax.fori_loop(0, B // L, scale, 0)
        pltpu.sync_copy(rows_v, agg_sh.at[dst_v.at[j]], add=True)
        return c
    lax.fori_loop(0, CH, body, 0)

    plsc.subcore_barrier()
    sl = pl.ds(sid * RPT, RPT)
    pltpu.sync_copy(agg_sh.at[sl], out_h.at[cid, sl])


def _mlp_in_call(x_p, w, b2):
    def body(x_r, w_r, b_r, o_r):
        o_r[...] = jnp.maximum(x_r[...] @ w_r[...] + b_r[...], 0.0)
    return pl.pallas_call(
        body,
        grid=(N_PAD // ROWBLK,),
        in_specs=[
            pl.BlockSpec((ROWBLK, IN_CH), lambda i: (i, 0)),
            pl.BlockSpec((IN_CH, D), lambda i: (0, 0)),
            pl.BlockSpec((1, D), lambda i: (0, 0)),
        ],
        out_specs=pl.BlockSpec((ROWBLK, D), lambda i: (i, 0)),
        out_shape=jax.ShapeDtypeStruct((N_PAD, D), jnp.float32),
    )(x_p, w, b2)


_DEG_ROWS = 2 * 2 * N_PAD // 128
_NR = N_PAD // 128


def _norm_call(deg_flat):
    def body(d_r, o_r):
        d = d_r[...]
        tot_o = d[0:_NR] + d[2 * _NR:3 * _NR]
        tot_i = d[_NR:2 * _NR] + d[3 * _NR:4 * _NR]
        no = jnp.where(tot_o > 0, lax.rsqrt(tot_o), 0.0)
        ni = jnp.where(tot_i > 0, lax.rsqrt(tot_i), 0.0)
        o_r[...] = jnp.concatenate([no, ni], axis=0)
    return pl.pallas_call(
        body,
        out_shape=jax.ShapeDtypeStruct((2 * _NR, 128), jnp.float32),
    )(deg_flat)


def _combine_call(part, feat0):
    def body(p_r, f_r, o_r):
        p = p_r[...]
        o_r[...] = (1.0 - ALPHA) * (p[0] + p[1]) + ALPHA * f_r[...]
    return pl.pallas_call(
        body,
        grid=(N_PAD // ROWBLK,),
        in_specs=[
            pl.BlockSpec((NC, ROWBLK, D), lambda i: (0, i, 0)),
            pl.BlockSpec((ROWBLK, D), lambda i: (i, 0)),
        ],
        out_specs=pl.BlockSpec((ROWBLK, D), lambda i: (i, 0)),
        out_shape=jax.ShapeDtypeStruct((N_PAD, D), jnp.float32),
    )(part, feat0)


def _final_call(part, feat0, w, b2):
    def body(p_r, f_r, w_r, b_r, o_r):
        p = p_r[...]
        h = (1.0 - ALPHA) * (p[0] + p[1]) + ALPHA * f_r[...]
        o_r[...] = h @ w_r[...] + b_r[...]
    return pl.pallas_call(
        body,
        grid=(N_PAD // ROWBLK,),
        in_specs=[
            pl.BlockSpec((NC, ROWBLK, D), lambda i: (0, i, 0)),
            pl.BlockSpec((ROWBLK, D), lambda i: (i, 0)),
            pl.BlockSpec((D, D), lambda i: (0, 0)),
            pl.BlockSpec((1, D), lambda i: (0, 0)),
        ],
        out_specs=pl.BlockSpec((ROWBLK, D), lambda i: (i, 0)),
        out_shape=jax.ShapeDtypeStruct((N_PAD, D), jnp.float32),
    )(part, feat0, w, b2)


def kernel(x, edge_index, edge_weight, W_in, b_in, W_out, b_out):
    src = edge_index[0]
    dst = edge_index[1]
    pad_idx = jnp.full((E_PAD - E,), N, jnp.int32)
    src_p = jnp.concatenate([src, pad_idx]).reshape(NW, CH, B)
    dst_p = jnp.concatenate([dst, pad_idx]).reshape(NW, CH, B)
    ew_p = jnp.concatenate(
        [edge_weight, jnp.zeros((E_PAD - E,), jnp.float32)]).reshape(NW, CH, B)
    x_p = jnp.pad(x, ((0, N_PAD - N), (0, 0)))

    deg = _deg_kernel(src_p, dst_p)
    norms = _norm_call(deg.reshape(_DEG_ROWS, 128))
    no = norms[:_NR].reshape(N_PAD)
    ni = norms[_NR:].reshape(N_PAD)

    h0 = _mlp_in_call(x_p, W_in, b_in.reshape(1, D))
    coef = _coef_kernel(src_p, dst_p, ew_p, no, ni)

    h = h0
    out = None
    for t in range(K):
        part = _prop_kernel(h, src_p, dst_p, coef)
        if t < K - 1:
            h = _combine_call(part, h0)
        else:
            out = _final_call(part, h0, W_out, b_out.reshape(1, D))
    return out[:N]
